# baseline (speedup 1.0000x reference)
import jax
import jax.numpy as jnp
from jax import lax
from jax.experimental import pallas as pl
from jax.experimental.pallas import tpu as pltpu
from jax.experimental.pallas import tpu_sc as plsc

N = 10000
E = 320000
IN_DIM = 128
OUT_DIM = 64
HEADS = 4

NS = 16
L = 16
EPT = E // NS
K = 80
NCHUNK = EPT // K
N2 = 10240
ROWS_PT = N2 // NS
EX1_OFF = 96
EXZ_OFF = 176
EXV_LEN = 192


def _tc_proj_body(h_ref, w_ref, u_ref, z3_ref, st_ref, m_ref):
    hb = h_ref[...]
    z = lax.dot_general(hb, w_ref[...], (((1,), (1,)), ((), ())),
                        preferred_element_type=jnp.float32)
    z3_ref[0] = z[:, :IN_DIM]
    z3_ref[1] = z[:, IN_DIM:]
    st = lax.dot_general(z, u_ref[...], (((1,), (0,)), ((), ())),
                         preferred_element_type=jnp.float32)
    st_ref[...] = st
    mcur = jnp.broadcast_to(jnp.max(st, axis=0)[:2 * HEADS, None],
                            (2 * HEADS, IN_DIM))
    i = pl.program_id(0)
    m_ref[...] = jnp.where(i == 0, mcur, jnp.maximum(m_ref[...], mcur))


def _tc_project(h, wcat, u16):
    R = 1000
    return pl.pallas_call(
        _tc_proj_body,
        grid=(N // R,),
        in_specs=[
            pl.BlockSpec((R, IN_DIM), lambda i: (i, 0)),
            pl.BlockSpec((HEADS * OUT_DIM, IN_DIM), lambda i: (0, 0)),
            pl.BlockSpec((HEADS * OUT_DIM, L), lambda i: (0, 0)),
        ],
        out_specs=[
            pl.BlockSpec((2, R, IN_DIM), lambda i: (0, i, 0)),
            pl.BlockSpec((R, L), lambda i: (i, 0)),
            pl.BlockSpec((2 * HEADS, IN_DIM), lambda i: (0, 0)),
        ],
        out_shape=[
            jax.ShapeDtypeStruct((2, N, IN_DIM), jnp.float32),
            jax.ShapeDtypeStruct((N, L), jnp.float32),
            jax.ShapeDtypeStruct((2 * HEADS, IN_DIM), jnp.float32),
        ],
    )(h, wcat, u16)


def _tc_norm_body(oa_ref, ob_ref, d0a_ref, d1a_ref, d0b_ref, d1b_ref, out_ref):
    def recip(dref):
        return 1.0 / jnp.maximum(
            jnp.sum(dref[...], axis=1, keepdims=True), 1e-30)
    r0a, r1a, r0b, r1b = recip(d0a_ref), recip(d1a_ref), recip(d0b_ref), recip(d1b_ref)
    out_ref[:, 0:OUT_DIM] = oa_ref[:, 0:OUT_DIM] * r0a
    out_ref[:, OUT_DIM:2 * OUT_DIM] = oa_ref[:, OUT_DIM:] * r1a
    out_ref[:, 2 * OUT_DIM:3 * OUT_DIM] = ob_ref[:, 0:OUT_DIM] * r0b
    out_ref[:, 3 * OUT_DIM:] = ob_ref[:, OUT_DIM:] * r1b


def _tc_normalize(oa, ob, dta, dtb):
    R = 1024
    nb = N2 // R
    return pl.pallas_call(
        _tc_norm_body,
        grid=(nb,),
        in_specs=[
            pl.BlockSpec((R, IN_DIM), lambda i: (i, 0)),
            pl.BlockSpec((R, IN_DIM), lambda i: (i, 0)),
            pl.BlockSpec((R, NS), lambda i: (i, 0)),
            pl.BlockSpec((R, NS), lambda i: (i + nb, 0)),
            pl.BlockSpec((R, NS), lambda i: (i, 0)),
            pl.BlockSpec((R, NS), lambda i: (i + nb, 0)),
        ],
        out_specs=pl.BlockSpec((R, HEADS * OUT_DIM), lambda i: (i, 0)),
        out_shape=jax.ShapeDtypeStruct((N2, HEADS * OUT_DIM), jnp.float32),
    )(oa, ob, dta, dta, dtb, dtb)


def _sc_body(z3, sflat, src_hbm, dst_hbm, m_hbm,
             out3, denp,
             ixs0, ixd0, ixs1, ixd1,
             si00, si01, si02, si03, si10, si11, si12, si13,
             sg00, sg01, sg02, sg03, sg10, sg11, sg12, sg13,
             rows_v0, rows_v1, den_local, exv, mall_v, ixb_s, ixb_d,
             semr0, sems0, semr1, sems1, semi, semw0, semw1,
             hout):
    c = lax.axis_index("c")
    s = lax.axis_index("s")
    z16 = jnp.zeros((L,), jnp.float32)
    lanes = lax.iota(jnp.int32, L)
    is0 = c == 0

    pltpu.sync_copy(m_hbm, mall_v)

    def _mhead(j):
        va = mall_v[j, pl.ds(0, L)] + mall_v[HEADS + j, pl.ds(0, L)]
        vb = mall_v[2 + j, pl.ds(0, L)] + mall_v[HEADS + 2 + j, pl.ds(0, L)]
        msum = jnp.where(is0, va, vb)
        return jnp.max(jnp.maximum(msum, 0.01 * msum))
    m0 = _mhead(0)
    m1 = _mhead(1)

    bufs = [
        (ixs0, ixd0, [si00, si01, si02, si03], [sg00, sg01, sg02, sg03],
         rows_v0, semr0, sems0),
        (ixs1, ixd1, [si10, si11, si12, si13], [sg10, sg11, sg12, sg13],
         rows_v1, semr1, sems1),
    ]
    semw = (semw0, semw1)

    def _zero_bufs(r, _):
        for j in range(IN_DIM // L):
            rows_v0[r, pl.ds(j * L, L)] = z16
        return 0
    lax.fori_loop(0, K, _zero_bufs, 0)

    def _zero_den(i, _):
        den_local[pl.ds(i * L, L)] = z16
        return 0
    lax.fori_loop(0, 2 * N2 // L, _zero_den, 0)

    for t in range(ROWS_PT // K):
        base = s * ROWS_PT + t * K
        pltpu.sync_copy(rows_v0, hout.at[pl.ds(base, K)])

    plsc.subcore_barrier()

    hoff = jnp.where(is0, 0, 2 * N)
    offs = (hoff, hoff + N, hoff + HEADS * N, hoff + (HEADS + 1) * N)

    def _issue(b):
        ixs, ixd, sis, sgs, rv, semr, sems = bufs[b]
        for g in range(K // L):
            ixs[pl.ds(g * L, L)] = ixb_s[pl.ds(b * K + g * L, L)]
            ixd[pl.ds(g * L, L)] = ixb_d[pl.ds(b * K + g * L, L)]
        pltpu.async_copy(z3.at[c].at[ixs], rv, semr)
        srcs = (ixs, ixs, ixd, ixd)
        for j in range(4):
            for g in range(K // L):
                sis[j][pl.ds(g * L, L)] = srcs[j][pl.ds(g * L, L)] + offs[j]
            pltpu.async_copy(sflat.at[sis[j]], sgs[j], sems)

    def _load_ixb(ch2, sync=False):
        ebase = s * EPT + ch2 * K
        if sync:
            pltpu.sync_copy(src_hbm.at[pl.ds(ebase, 2 * K)], ixb_s)
            pltpu.sync_copy(dst_hbm.at[pl.ds(ebase, 2 * K)], ixb_d)
        else:
            pltpu.async_copy(src_hbm.at[pl.ds(ebase, 2 * K)], ixb_s, semi)
            pltpu.async_copy(dst_hbm.at[pl.ds(ebase, 2 * K)], ixb_d, semi)

    def _wait_ixb():
        pltpu.make_async_copy(src_hbm.at[pl.ds(0, 2 * K)], ixb_s, semi).wait()
        pltpu.make_async_copy(dst_hbm.at[pl.ds(0, 2 * K)], ixb_d, semi).wait()

    def _process(b):
        ixs, ixd, sis, sgs, rv, semr, sems = bufs[b]
        pltpu.make_async_copy(z3.at[c].at[ixs], rv, semr).wait()
        for j in range(4):
            pltpu.make_async_copy(sflat.at[sis[j]], sgs[j], sems).wait()
        sg0, sg1, dg0, dg1 = sgs
        for g in range(K // L):
            e0 = sg0[pl.ds(g * L, L)] + dg0[pl.ds(g * L, L)]
            e1 = sg1[pl.ds(g * L, L)] + dg1[pl.ds(g * L, L)]
            e0 = jnp.maximum(e0, 0.01 * e0)
            e1 = jnp.maximum(e1, 0.01 * e1)
            ex0 = jnp.exp(e0 - m0)
            ex1 = jnp.exp(e1 - m1)
            exv[pl.ds(g * L, L)] = ex0
            exv[pl.ds(EX1_OFF + g * L, L)] = ex1
            dstv = ixd[pl.ds(g * L, L)]
            plsc.addupdate_scatter(den_local, [dstv], ex0)
            plsc.addupdate_scatter(den_local, [dstv + N2], ex1)

        def _scale(r, _):
            rr = jnp.full((L,), r, jnp.int32)
            w0 = plsc.load_gather(exv, [rr])
            w1 = plsc.load_gather(exv, [rr + EX1_OFF])
            for j in range(IN_DIM // L):
                w = w0 if j < (IN_DIM // L) // 2 else w1
                rv[r, pl.ds(j * L, L)] = rv[r, pl.ds(j * L, L)] * w
            return 0
        lax.fori_loop(0, K, _scale, 0, unroll=4)
        pltpu.async_copy(rv, hout.at[ixd], semw[b], add=True)

    def _wait_scatter(b):
        ixs, ixd, sis, sgs, rv, semr, sems = bufs[b]
        pltpu.make_async_copy(rv, hout.at[ixd], semw[b]).wait()

    _load_ixb(0, sync=True)
    _issue(0)
    _issue(1)
    _load_ixb(2)

    def _pair(i, _):
        _process(0)
        _process(1)

        @pl.when(i < NCHUNK // 2 - 1)
        def _():
            _wait_ixb()
            _wait_scatter(0)
            _issue(0)
            _wait_scatter(1)
            _issue(1)

        @pl.when(i < NCHUNK // 2 - 2)
        def _():
            _load_ixb(2 * i + 4)

        return 0

    lax.fori_loop(0, NCHUNK // 2, _pair, 0)
    _wait_scatter(0)
    _wait_scatter(1)
    plsc.subcore_barrier()

    rbase = s * ROWS_PT
    pltpu.sync_copy(hout.at[pl.ds(rbase, ROWS_PT)],
                    out3.at[c].at[pl.ds(rbase, ROWS_PT)])
    pltpu.sync_copy(den_local, denp.at[c].at[s])


def _build_sc():
    mesh = plsc.VectorSubcoreMesh(core_axis_name="c", subcore_axis_name="s")
    return pl.kernel(
        _sc_body,
        out_type=[
            jax.ShapeDtypeStruct((2, N2, IN_DIM), jnp.float32),
            jax.ShapeDtypeStruct((2, NS, 2 * N2), jnp.float32),
        ],
        mesh=mesh,
        compiler_params=pltpu.CompilerParams(needs_layout_passes=False),
        scratch_types=(
            [pltpu.VMEM((K,), jnp.int32) for _ in range(4)]
            + [pltpu.VMEM((K,), jnp.int32) for _ in range(8)]
            + [pltpu.VMEM((K,), jnp.float32) for _ in range(8)]
            + [
                pltpu.VMEM((K, IN_DIM), jnp.float32),
                pltpu.VMEM((K, IN_DIM), jnp.float32),
                pltpu.VMEM((2 * N2,), jnp.float32),
                pltpu.VMEM((EXV_LEN,), jnp.float32),
                pltpu.VMEM((2 * HEADS, IN_DIM), jnp.float32),
                pltpu.VMEM((2 * K,), jnp.int32),
                pltpu.VMEM((2 * K,), jnp.int32),
                pltpu.SemaphoreType.DMA,
                pltpu.SemaphoreType.DMA,
                pltpu.SemaphoreType.DMA,
                pltpu.SemaphoreType.DMA,
                pltpu.SemaphoreType.DMA,
                pltpu.SemaphoreType.DMA,
                pltpu.SemaphoreType.DMA,
                pltpu.VMEM_SHARED((N2, IN_DIM), jnp.float32),
            ]
        ),
    )


@jax.jit
def kernel(h, edge_index, W, A):
    src = edge_index[0]
    dst = edge_index[1]
    wcat = W.reshape(HEADS * OUT_DIM, IN_DIM)
    eye = jnp.eye(HEADS, dtype=jnp.float32)
    u_src = jnp.einsum("hg,hk->hgk", eye, A[:, :OUT_DIM]).reshape(HEADS, HEADS * OUT_DIM)
    u_dst = jnp.einsum("hg,hk->hgk", eye, A[:, OUT_DIM:]).reshape(HEADS, HEADS * OUT_DIM)
    u16 = jnp.zeros((HEADS * OUT_DIM, L), jnp.float32)
    u16 = u16.at[:, :2 * HEADS].set(jnp.concatenate([u_src, u_dst], axis=0).T)

    z3, st16, m = _tc_project(h, wcat, u16)
    sflat = st16[:, :2 * HEADS].T.reshape(-1)

    out3, denp = _build_sc()(z3, sflat, src, dst, m)
    dta = denp[0].T
    dtb = denp[1].T
    out = _tc_normalize(out3[0], out3[1], dta, dtb)
    return out[:N]


# speedup vs baseline: 1.0794x; 1.0794x over previous
import jax
import jax.numpy as jnp
from jax import lax
from jax.experimental import pallas as pl
from jax.experimental.pallas import tpu as pltpu
from jax.experimental.pallas import tpu_sc as plsc

N = 10000
E = 320000
IN_DIM = 128
OUT_DIM = 64
HEADS = 4

NS = 16
L = 16
EPT = E // NS
K = 80
NCHUNK = EPT // K
N2 = 10240
ROWS_PT = N2 // NS
EX1_OFF = 96
EXZ_OFF = 176
EXV_LEN = 192


def _tc_proj_body(h_ref, w_ref, u_ref, z3_ref, st_ref, m_ref):
    hb = h_ref[...]
    z = lax.dot_general(hb, w_ref[...], (((1,), (1,)), ((), ())),
                        preferred_element_type=jnp.float32)
    z3_ref[0] = z[:, :IN_DIM]
    z3_ref[1] = z[:, IN_DIM:]
    st = lax.dot_general(z, u_ref[...], (((1,), (0,)), ((), ())),
                         preferred_element_type=jnp.float32)
    st_ref[...] = st
    mcur = jnp.broadcast_to(jnp.max(st, axis=0)[:2 * HEADS, None],
                            (2 * HEADS, IN_DIM))
    i = pl.program_id(0)
    m_ref[...] = jnp.where(i == 0, mcur, jnp.maximum(m_ref[...], mcur))


def _tc_project(h, wcat, u16):
    R = 1000
    return pl.pallas_call(
        _tc_proj_body,
        grid=(N // R,),
        in_specs=[
            pl.BlockSpec((R, IN_DIM), lambda i: (i, 0)),
            pl.BlockSpec((HEADS * OUT_DIM, IN_DIM), lambda i: (0, 0)),
            pl.BlockSpec((HEADS * OUT_DIM, L), lambda i: (0, 0)),
        ],
        out_specs=[
            pl.BlockSpec((2, R, IN_DIM), lambda i: (0, i, 0)),
            pl.BlockSpec((R, L), lambda i: (i, 0)),
            pl.BlockSpec((2 * HEADS, IN_DIM), lambda i: (0, 0)),
        ],
        out_shape=[
            jax.ShapeDtypeStruct((2, N, IN_DIM), jnp.float32),
            jax.ShapeDtypeStruct((N, L), jnp.float32),
            jax.ShapeDtypeStruct((2 * HEADS, IN_DIM), jnp.float32),
        ],
    )(h, wcat, u16)


def _tc_norm_body(oa_ref, ob_ref, d0a_ref, d1a_ref, d0b_ref, d1b_ref, out_ref):
    def recip(dref):
        return 1.0 / jnp.maximum(
            jnp.sum(dref[...], axis=1, keepdims=True), 1e-30)
    r0a, r1a, r0b, r1b = recip(d0a_ref), recip(d1a_ref), recip(d0b_ref), recip(d1b_ref)
    out_ref[:, 0:OUT_DIM] = oa_ref[:, 0:OUT_DIM] * r0a
    out_ref[:, OUT_DIM:2 * OUT_DIM] = oa_ref[:, OUT_DIM:] * r1a
    out_ref[:, 2 * OUT_DIM:3 * OUT_DIM] = ob_ref[:, 0:OUT_DIM] * r0b
    out_ref[:, 3 * OUT_DIM:] = ob_ref[:, OUT_DIM:] * r1b


def _tc_normalize(oa, ob, dta, dtb):
    R = 1024
    nb = N2 // R
    return pl.pallas_call(
        _tc_norm_body,
        grid=(nb,),
        in_specs=[
            pl.BlockSpec((R, IN_DIM), lambda i: (i, 0)),
            pl.BlockSpec((R, IN_DIM), lambda i: (i, 0)),
            pl.BlockSpec((R, NS), lambda i: (i, 0)),
            pl.BlockSpec((R, NS), lambda i: (i + nb, 0)),
            pl.BlockSpec((R, NS), lambda i: (i, 0)),
            pl.BlockSpec((R, NS), lambda i: (i + nb, 0)),
        ],
        out_specs=pl.BlockSpec((R, HEADS * OUT_DIM), lambda i: (i, 0)),
        out_shape=jax.ShapeDtypeStruct((N2, HEADS * OUT_DIM), jnp.float32),
    )(oa, ob, dta, dta, dtb, dtb)


def _sc_body(z3, sflat, src_hbm, dst_hbm, m_hbm,
             out3, denp,
             ixs0, ixd0, ixs1, ixd1,
             si00, si01, si02, si03, si10, si11, si12, si13,
             sg00, sg01, sg02, sg03, sg10, sg11, sg12, sg13,
             rows_v0, rows_v1, den_local, exv, mall_v, ixb_s, ixb_d,
             semr0, sems0, semr1, sems1, semi,
             hout):
    c = lax.axis_index("c")
    s = lax.axis_index("s")
    z16 = jnp.zeros((L,), jnp.float32)
    lanes = lax.iota(jnp.int32, L)
    is0 = c == 0

    pltpu.sync_copy(m_hbm, mall_v)

    def _mhead(j):
        va = mall_v[j, pl.ds(0, L)] + mall_v[HEADS + j, pl.ds(0, L)]
        vb = mall_v[2 + j, pl.ds(0, L)] + mall_v[HEADS + 2 + j, pl.ds(0, L)]
        msum = jnp.where(is0, va, vb)
        return jnp.max(jnp.maximum(msum, 0.01 * msum))
    m0 = _mhead(0)
    m1 = _mhead(1)

    bufs = [
        (ixs0, ixd0, [si00, si01, si02, si03], [sg00, sg01, sg02, sg03],
         rows_v0, semr0, sems0),
        (ixs1, ixd1, [si10, si11, si12, si13], [sg10, sg11, sg12, sg13],
         rows_v1, semr1, sems1),
    ]

    def _zero_bufs(r, _):
        for j in range(IN_DIM // L):
            rows_v0[r, pl.ds(j * L, L)] = z16
        return 0
    lax.fori_loop(0, K, _zero_bufs, 0)

    def _zero_den(i, _):
        den_local[pl.ds(i * L, L)] = z16
        return 0
    lax.fori_loop(0, 2 * N2 // L, _zero_den, 0)

    for t in range(ROWS_PT // K):
        base = s * ROWS_PT + t * K
        pltpu.sync_copy(rows_v0, hout.at[pl.ds(base, K)])

    plsc.subcore_barrier()

    hoff = jnp.where(is0, 0, 2 * N)
    offs = (hoff, hoff + N, hoff + HEADS * N, hoff + (HEADS + 1) * N)

    def _issue(b):
        ixs, ixd, sis, sgs, rv, semr, sems = bufs[b]
        for g in range(K // L):
            ixs[pl.ds(g * L, L)] = ixb_s[pl.ds(b * K + g * L, L)]
            ixd[pl.ds(g * L, L)] = ixb_d[pl.ds(b * K + g * L, L)]
        pltpu.async_copy(z3.at[c].at[ixs], rv, semr)
        srcs = (ixs, ixs, ixd, ixd)
        for j in range(4):
            for g in range(K // L):
                sis[j][pl.ds(g * L, L)] = srcs[j][pl.ds(g * L, L)] + offs[j]
            pltpu.async_copy(sflat.at[sis[j]], sgs[j], sems)

    def _load_ixb(ch2, sync=False):
        ebase = s * EPT + ch2 * K
        if sync:
            pltpu.sync_copy(src_hbm.at[pl.ds(ebase, 2 * K)], ixb_s)
            pltpu.sync_copy(dst_hbm.at[pl.ds(ebase, 2 * K)], ixb_d)
        else:
            pltpu.async_copy(src_hbm.at[pl.ds(ebase, 2 * K)], ixb_s, semi)
            pltpu.async_copy(dst_hbm.at[pl.ds(ebase, 2 * K)], ixb_d, semi)

    def _wait_ixb():
        pltpu.make_async_copy(src_hbm.at[pl.ds(0, 2 * K)], ixb_s, semi).wait()
        pltpu.make_async_copy(dst_hbm.at[pl.ds(0, 2 * K)], ixb_d, semi).wait()

    def _process(b):
        ixs, ixd, sis, sgs, rv, semr, sems = bufs[b]
        pltpu.make_async_copy(z3.at[c].at[ixs], rv, semr).wait()
        for j in range(4):
            pltpu.make_async_copy(sflat.at[sis[j]], sgs[j], sems).wait()
        sg0, sg1, dg0, dg1 = sgs
        for g in range(K // L):
            e0 = sg0[pl.ds(g * L, L)] + dg0[pl.ds(g * L, L)]
            e1 = sg1[pl.ds(g * L, L)] + dg1[pl.ds(g * L, L)]
            e0 = jnp.maximum(e0, 0.01 * e0)
            e1 = jnp.maximum(e1, 0.01 * e1)
            ex0 = jnp.exp(e0 - m0)
            ex1 = jnp.exp(e1 - m1)
            exv[pl.ds(g * L, L)] = ex0
            exv[pl.ds(EX1_OFF + g * L, L)] = ex1
            dstv = ixd[pl.ds(g * L, L)]
            plsc.addupdate_scatter(den_local, [dstv], ex0)
            plsc.addupdate_scatter(den_local, [dstv + N2], ex1)

        def _scale(r, _):
            rr = jnp.full((L,), r, jnp.int32)
            w0 = plsc.load_gather(exv, [rr])
            w1 = plsc.load_gather(exv, [rr + EX1_OFF])
            for j in range(IN_DIM // L):
                w = w0 if j < (IN_DIM // L) // 2 else w1
                rv[r, pl.ds(j * L, L)] = rv[r, pl.ds(j * L, L)] * w
            return 0
        lax.fori_loop(0, K, _scale, 0, unroll=4)
        pltpu.sync_copy(rv, hout.at[ixd], add=True)

    _load_ixb(0, sync=True)
    _issue(0)
    _issue(1)
    _load_ixb(2)

    def _pair(i, _):
        _process(0)

        @pl.when(i < NCHUNK // 2 - 1)
        def _():
            _wait_ixb()
            _issue(0)

        _process(1)

        @pl.when(i < NCHUNK // 2 - 1)
        def _():
            _issue(1)

        @pl.when(i < NCHUNK // 2 - 2)
        def _():
            _load_ixb(2 * i + 4)

        return 0

    lax.fori_loop(0, NCHUNK // 2, _pair, 0)
    plsc.subcore_barrier()

    rbase = s * ROWS_PT
    pltpu.sync_copy(hout.at[pl.ds(rbase, ROWS_PT)],
                    out3.at[c].at[pl.ds(rbase, ROWS_PT)])
    pltpu.sync_copy(den_local, denp.at[c].at[s])


def _build_sc():
    mesh = plsc.VectorSubcoreMesh(core_axis_name="c", subcore_axis_name="s")
    return pl.kernel(
        _sc_body,
        out_type=[
            jax.ShapeDtypeStruct((2, N2, IN_DIM), jnp.float32),
            jax.ShapeDtypeStruct((2, NS, 2 * N2), jnp.float32),
        ],
        mesh=mesh,
        compiler_params=pltpu.CompilerParams(needs_layout_passes=False),
        scratch_types=(
            [pltpu.VMEM((K,), jnp.int32) for _ in range(4)]
            + [pltpu.VMEM((K,), jnp.int32) for _ in range(8)]
            + [pltpu.VMEM((K,), jnp.float32) for _ in range(8)]
            + [
                pltpu.VMEM((K, IN_DIM), jnp.float32),
                pltpu.VMEM((K, IN_DIM), jnp.float32),
                pltpu.VMEM((2 * N2,), jnp.float32),
                pltpu.VMEM((EXV_LEN,), jnp.float32),
                pltpu.VMEM((2 * HEADS, IN_DIM), jnp.float32),
                pltpu.VMEM((2 * K,), jnp.int32),
                pltpu.VMEM((2 * K,), jnp.int32),
                pltpu.SemaphoreType.DMA,
                pltpu.SemaphoreType.DMA,
                pltpu.SemaphoreType.DMA,
                pltpu.SemaphoreType.DMA,
                pltpu.SemaphoreType.DMA,
                pltpu.VMEM_SHARED((N2, IN_DIM), jnp.float32),
            ]
        ),
    )


@jax.jit
def kernel(h, edge_index, W, A):
    src = edge_index[0]
    dst = edge_index[1]
    wcat = W.reshape(HEADS * OUT_DIM, IN_DIM)
    eye = jnp.eye(HEADS, dtype=jnp.float32)
    u_src = jnp.einsum("hg,hk->hgk", eye, A[:, :OUT_DIM]).reshape(HEADS, HEADS * OUT_DIM)
    u_dst = jnp.einsum("hg,hk->hgk", eye, A[:, OUT_DIM:]).reshape(HEADS, HEADS * OUT_DIM)
    u16 = jnp.zeros((HEADS * OUT_DIM, L), jnp.float32)
    u16 = u16.at[:, :2 * HEADS].set(jnp.concatenate([u_src, u_dst], axis=0).T)

    z3, st16, m = _tc_project(h, wcat, u16)
    sflat = st16[:, :2 * HEADS].T.reshape(-1)

    out3, denp = _build_sc()(z3, sflat, src, dst, m)
    dta = denp[0].T
    dtb = denp[1].T
    out = _tc_normalize(out3[0], out3[1], dta, dtb)
    return out[:N]


# speedup vs baseline: 1.0801x; 1.0007x over previous
import jax
import jax.numpy as jnp
from jax import lax
from jax.experimental import pallas as pl
from jax.experimental.pallas import tpu as pltpu
from jax.experimental.pallas import tpu_sc as plsc

N = 10000
E = 320000
IN_DIM = 128
OUT_DIM = 64
HEADS = 4

NS = 16
L = 16
EPT = E // NS
K = 80
NCHUNK = EPT // K
N2 = 10240
ROWS_PT = N2 // NS
EX1_OFF = 96
EXV_LEN = 192


def _tc_proj_body(h_ref, w_ref, u_ref, z3_ref, st_ref, m_ref):
    hb = h_ref[...]
    z = lax.dot_general(hb, w_ref[...], (((1,), (1,)), ((), ())),
                        preferred_element_type=jnp.float32)
    z3_ref[0] = z[:, :IN_DIM]
    z3_ref[1] = z[:, IN_DIM:]
    st = lax.dot_general(z, u_ref[...], (((1,), (0,)), ((), ())),
                         preferred_element_type=jnp.float32)
    st_ref[...] = st
    mcur = jnp.broadcast_to(jnp.max(st, axis=0)[:2 * HEADS, None],
                            (2 * HEADS, IN_DIM))
    i = pl.program_id(0)
    m_ref[...] = jnp.where(i == 0, mcur, jnp.maximum(m_ref[...], mcur))


def _tc_project(h, wcat, u16):
    R = 1000
    return pl.pallas_call(
        _tc_proj_body,
        grid=(N // R,),
        in_specs=[
            pl.BlockSpec((R, IN_DIM), lambda i: (i, 0)),
            pl.BlockSpec((HEADS * OUT_DIM, IN_DIM), lambda i: (0, 0)),
            pl.BlockSpec((HEADS * OUT_DIM, L), lambda i: (0, 0)),
        ],
        out_specs=[
            pl.BlockSpec((2, R, IN_DIM), lambda i: (0, i, 0)),
            pl.BlockSpec((R, L), lambda i: (i, 0)),
            pl.BlockSpec((2 * HEADS, IN_DIM), lambda i: (0, 0)),
        ],
        out_shape=[
            jax.ShapeDtypeStruct((2, N, IN_DIM), jnp.float32),
            jax.ShapeDtypeStruct((N, L), jnp.float32),
            jax.ShapeDtypeStruct((2 * HEADS, IN_DIM), jnp.float32),
        ],
    )(h, wcat, u16)


def _tc_norm_body(oa_ref, ob_ref, d0a_ref, d1a_ref, d0b_ref, d1b_ref, out_ref):
    def recip(dref):
        return 1.0 / jnp.maximum(
            jnp.sum(dref[...], axis=1, keepdims=True), 1e-30)
    r0a, r1a, r0b, r1b = recip(d0a_ref), recip(d1a_ref), recip(d0b_ref), recip(d1b_ref)
    out_ref[:, 0:OUT_DIM] = oa_ref[:, 0:OUT_DIM] * r0a
    out_ref[:, OUT_DIM:2 * OUT_DIM] = oa_ref[:, OUT_DIM:] * r1a
    out_ref[:, 2 * OUT_DIM:3 * OUT_DIM] = ob_ref[:, 0:OUT_DIM] * r0b
    out_ref[:, 3 * OUT_DIM:] = ob_ref[:, OUT_DIM:] * r1b


def _tc_normalize(oa, ob, dta, dtb):
    R = 1024
    nb = N2 // R
    return pl.pallas_call(
        _tc_norm_body,
        grid=(nb,),
        in_specs=[
            pl.BlockSpec((R, IN_DIM), lambda i: (i, 0)),
            pl.BlockSpec((R, IN_DIM), lambda i: (i, 0)),
            pl.BlockSpec((R, NS), lambda i: (i, 0)),
            pl.BlockSpec((R, NS), lambda i: (i + nb, 0)),
            pl.BlockSpec((R, NS), lambda i: (i, 0)),
            pl.BlockSpec((R, NS), lambda i: (i + nb, 0)),
        ],
        out_specs=pl.BlockSpec((R, HEADS * OUT_DIM), lambda i: (i, 0)),
        out_shape=jax.ShapeDtypeStruct((N2, HEADS * OUT_DIM), jnp.float32),
    )(oa, ob, dta, dta, dtb, dtb)


def _sc_body(z3, sflat, src_hbm, dst_hbm, m_hbm,
             out3, denp,
             ixs0, ixd0, ixs1, ixd1,
             si00, si01, si02, si03, si10, si11, si12, si13,
             sg00, sg01, sg02, sg03, sg10, sg11, sg12, sg13,
             rows_v0, rows_v1, den_local, exv, mall_v, ixb_s, ixb_d,
             semr0, sems0, semr1, sems1, semi,
             hout):
    c = lax.axis_index("c")
    s = lax.axis_index("s")
    z16 = jnp.zeros((L,), jnp.float32)
    is0 = c == 0

    pltpu.sync_copy(m_hbm, mall_v)

    def _mhead(j):
        va = mall_v[j, pl.ds(0, L)] + mall_v[HEADS + j, pl.ds(0, L)]
        vb = mall_v[2 + j, pl.ds(0, L)] + mall_v[HEADS + 2 + j, pl.ds(0, L)]
        msum = jnp.where(is0, va, vb)
        return jnp.max(jnp.maximum(msum, 0.01 * msum))
    m0 = _mhead(0)
    m1 = _mhead(1)

    bufs = [
        (ixs0, ixd0, [si00, si01, si02, si03], [sg00, sg01, sg02, sg03],
         rows_v0, semr0, sems0),
        (ixs1, ixd1, [si10, si11, si12, si13], [sg10, sg11, sg12, sg13],
         rows_v1, semr1, sems1),
    ]

    def _zero_bufs(r, _):
        for j in range(IN_DIM // L):
            rows_v0[r, pl.ds(j * L, L)] = z16
        return 0
    lax.fori_loop(0, K, _zero_bufs, 0)

    def _zero_den(i, _):
        den_local[pl.ds(i * L, L)] = z16
        return 0
    lax.fori_loop(0, 2 * N2 // L, _zero_den, 0)

    for t in range(ROWS_PT // K):
        base = s * ROWS_PT + t * K
        pltpu.sync_copy(rows_v0, hout.at[pl.ds(base, K)])

    plsc.subcore_barrier()

    hoff = jnp.where(is0, 0, 2 * N)
    offs = (hoff, hoff + N, hoff + HEADS * N, hoff + (HEADS + 1) * N)

    def _issue(b):
        ixs, ixd, sis, sgs, rv, semr, sems = bufs[b]
        for g in range(K // L):
            ixs[pl.ds(g * L, L)] = ixb_s[pl.ds(b * K + g * L, L)]
            ixd[pl.ds(g * L, L)] = ixb_d[pl.ds(b * K + g * L, L)]
        pltpu.async_copy(z3.at[c].at[ixs], rv, semr)
        srcs = (ixs, ixs, ixd, ixd)
        for j in range(4):
            for g in range(K // L):
                sis[j][pl.ds(g * L, L)] = srcs[j][pl.ds(g * L, L)] + offs[j]
            pltpu.async_copy(sflat.at[sis[j]], sgs[j], sems)

    def _load_ixb(ch2, sync=False):
        ebase = s * EPT + ch2 * K
        if sync:
            pltpu.sync_copy(src_hbm.at[pl.ds(ebase, 2 * K)], ixb_s)
            pltpu.sync_copy(dst_hbm.at[pl.ds(ebase, 2 * K)], ixb_d)
        else:
            pltpu.async_copy(src_hbm.at[pl.ds(ebase, 2 * K)], ixb_s, semi)
            pltpu.async_copy(dst_hbm.at[pl.ds(ebase, 2 * K)], ixb_d, semi)

    def _wait_ixb():
        pltpu.make_async_copy(src_hbm.at[pl.ds(0, 2 * K)], ixb_s, semi).wait()
        pltpu.make_async_copy(dst_hbm.at[pl.ds(0, 2 * K)], ixb_d, semi).wait()

    def _process(b):
        ixs, ixd, sis, sgs, rv, semr, sems = bufs[b]
        pltpu.make_async_copy(z3.at[c].at[ixs], rv, semr).wait()
        for j in range(4):
            pltpu.make_async_copy(sflat.at[sis[j]], sgs[j], sems).wait()
        sg0, sg1, dg0, dg1 = sgs
        for g in range(K // L):
            e0 = sg0[pl.ds(g * L, L)] + dg0[pl.ds(g * L, L)]
            e1 = sg1[pl.ds(g * L, L)] + dg1[pl.ds(g * L, L)]
            e0 = jnp.maximum(e0, 0.01 * e0)
            e1 = jnp.maximum(e1, 0.01 * e1)
            ex0 = jnp.exp(e0 - m0)
            ex1 = jnp.exp(e1 - m1)
            exv[pl.ds(g * L, L)] = ex0
            exv[pl.ds(EX1_OFF + g * L, L)] = ex1
            dstv = ixd[pl.ds(g * L, L)]
            plsc.addupdate_scatter(den_local, [dstv], ex0)
            plsc.addupdate_scatter(den_local, [dstv + N2], ex1)

        def _scale(r, _):
            rr = jnp.full((L,), r, jnp.int32)
            w0 = plsc.load_gather(exv, [rr])
            w1 = plsc.load_gather(exv, [rr + EX1_OFF])
            for j in range(IN_DIM // L):
                w = w0 if j < (IN_DIM // L) // 2 else w1
                rv[r, pl.ds(j * L, L)] = rv[r, pl.ds(j * L, L)] * w
            return 0
        lax.fori_loop(0, K, _scale, 0, unroll=4)
        pltpu.sync_copy(rv, hout.at[ixd], add=True)

    _load_ixb(0, sync=True)
    _issue(0)
    _issue(1)
    _load_ixb(2)

    def _pair(i, _):
        _process(0)

        @pl.when(i < NCHUNK // 2 - 1)
        def _():
            _wait_ixb()
            _issue(0)

        _process(1)

        @pl.when(i < NCHUNK // 2 - 1)
        def _():
            _issue(1)

        @pl.when(i < NCHUNK // 2 - 2)
        def _():
            _load_ixb(2 * i + 4)

        return 0

    lax.fori_loop(0, NCHUNK // 2, _pair, 0)
    plsc.subcore_barrier()

    rbase = s * ROWS_PT
    pltpu.sync_copy(hout.at[pl.ds(rbase, ROWS_PT)],
                    out3.at[c].at[pl.ds(rbase, ROWS_PT)])
    pltpu.sync_copy(den_local, denp.at[c].at[s])


def _build_sc():
    mesh = plsc.VectorSubcoreMesh(core_axis_name="c", subcore_axis_name="s")
    return pl.kernel(
        _sc_body,
        out_type=[
            jax.ShapeDtypeStruct((2, N2, IN_DIM), jnp.float32),
            jax.ShapeDtypeStruct((2, NS, 2 * N2), jnp.float32),
        ],
        mesh=mesh,
        compiler_params=pltpu.CompilerParams(needs_layout_passes=False),
        scratch_types=(
            [pltpu.VMEM((K,), jnp.int32) for _ in range(4)]
            + [pltpu.VMEM((K,), jnp.int32) for _ in range(8)]
            + [pltpu.VMEM((K,), jnp.float32) for _ in range(8)]
            + [
                pltpu.VMEM((K, IN_DIM), jnp.float32),
                pltpu.VMEM((K, IN_DIM), jnp.float32),
                pltpu.VMEM((2 * N2,), jnp.float32),
                pltpu.VMEM((EXV_LEN,), jnp.float32),
                pltpu.VMEM((2 * HEADS, IN_DIM), jnp.float32),
                pltpu.VMEM((2 * K,), jnp.int32),
                pltpu.VMEM((2 * K,), jnp.int32),
                pltpu.SemaphoreType.DMA,
                pltpu.SemaphoreType.DMA,
                pltpu.SemaphoreType.DMA,
                pltpu.SemaphoreType.DMA,
                pltpu.SemaphoreType.DMA,
                pltpu.VMEM_SHARED((N2, IN_DIM), jnp.float32),
            ]
        ),
    )


@jax.jit
def kernel(h, edge_index, W, A):
    src = edge_index[0]
    dst = edge_index[1]
    wcat = W.reshape(HEADS * OUT_DIM, IN_DIM)
    eye = jnp.eye(HEADS, dtype=jnp.float32)
    u_src = jnp.einsum("hg,hk->hgk", eye, A[:, :OUT_DIM]).reshape(HEADS, HEADS * OUT_DIM)
    u_dst = jnp.einsum("hg,hk->hgk", eye, A[:, OUT_DIM:]).reshape(HEADS, HEADS * OUT_DIM)
    u16 = jnp.zeros((HEADS * OUT_DIM, L), jnp.float32)
    u16 = u16.at[:, :2 * HEADS].set(jnp.concatenate([u_src, u_dst], axis=0).T)

    z3, st16, m = _tc_project(h, wcat, u16)
    sflat = st16[:, :2 * HEADS].T.reshape(-1)

    out3, denp = _build_sc()(z3, sflat, src, dst, m)
    dta = denp[0].T
    dtb = denp[1].T
    out = _tc_normalize(out3[0], out3[1], dta, dtb)
    return out[:N]
